# baseline (speedup 1.0000x reference)
import functools

import jax
import jax.numpy as jnp
from jax import lax
from jax.experimental import pallas as pl
from jax.experimental.pallas import tpu as pltpu
from jax.experimental.pallas import tpu_sc as plsc

NC = 2
NS = 16
NW = NC * NS
LANES = 16

C = 112
ZR = 64


def _sc_degree(dst3, n_pad, iters):
  sub_rows = n_pad // NS
  mesh = plsc.VectorSubcoreMesh(core_axis_name="c", subcore_axis_name="s")

  @functools.partial(
      pl.kernel,
      out_type=jax.ShapeDtypeStruct((NC, n_pad), jnp.float32),
      mesh=mesh,
      scratch_types=[
          pltpu.VMEM((iters, C), jnp.int32),
          pltpu.VMEM((C,), jnp.float32),
          pltpu.VMEM((sub_rows,), jnp.float32),
          pltpu.VMEM_SHARED((n_pad,), jnp.float32),
          pltpu.SemaphoreType.DMA,
      ],
  )
  def k(dst_hbm, out_hbm, idx_v, ones_v, zer_v, acc_sh, sem):
    cid = lax.axis_index("c")
    sid = lax.axis_index("s")
    wid = cid * NS + sid
    pltpu.async_copy(dst_hbm.at[wid], idx_v, sem)
    for i in range(C // LANES):
      ones_v[pl.ds(i * LANES, LANES)] = jnp.ones((LANES,), jnp.float32)
    for i in range(sub_rows // LANES):
      zer_v[pl.ds(i * LANES, LANES)] = jnp.zeros((LANES,), jnp.float32)
    pltpu.sync_copy(zer_v, acc_sh.at[pl.ds(sid * sub_rows, sub_rows)])
    pltpu.make_async_copy(dst_hbm.at[wid], idx_v, sem).wait()
    plsc.subcore_barrier()

    def body(i, carry):
      pltpu.sync_copy(ones_v, acc_sh.at[idx_v.at[i]], add=True)
      return carry

    lax.fori_loop(0, iters, body, 0)
    plsc.subcore_barrier()
    pltpu.sync_copy(
        acc_sh.at[pl.ds(sid * sub_rows, sub_rows)],
        out_hbm.at[cid, pl.ds(sid * sub_rows, sub_rows)],
    )

  return k(dst3)


def _sc_aggregate(g, src3, dst3, n_pad, iters, d):
  sub_rows = n_pad // NS
  mesh = plsc.VectorSubcoreMesh(core_axis_name="c", subcore_axis_name="s")

  @functools.partial(
      pl.kernel,
      out_type=jax.ShapeDtypeStruct((NC, n_pad, d), jnp.float32),
      mesh=mesh,
      scratch_types=[
          pltpu.VMEM((iters, C), jnp.int32),
          pltpu.VMEM((iters, C), jnp.int32),
          pltpu.VMEM((C, d), jnp.float32),
          pltpu.VMEM((C, d), jnp.float32),
          pltpu.VMEM_SHARED((n_pad, d), jnp.float32),
          pltpu.SemaphoreType.DMA,
          pltpu.SemaphoreType.DMA,
          pltpu.SemaphoreType.DMA,
          pltpu.SemaphoreType.DMA,
      ],
      compiler_params=pltpu.CompilerParams(use_tc_tiling_on_sc=False),
  )
  def k(g_hbm, src_hbm, dst_hbm, out_hbm, sidx, didx, buf0, buf1,
        acc_sh, sem0, sem1, ssem0, ssem1):
    cid = lax.axis_index("c")
    sid = lax.axis_index("s")
    wid = cid * NS + sid
    pltpu.async_copy(src_hbm.at[wid], sidx, sem0)
    pltpu.async_copy(dst_hbm.at[wid], didx, sem1)
    for r in range(ZR):
      for j in range(d // LANES):
        buf0[r, pl.ds(j * LANES, LANES)] = jnp.zeros((LANES,), jnp.float32)
    for b in range(sub_rows // ZR):
      pltpu.sync_copy(
          buf0.at[pl.ds(0, ZR)],
          acc_sh.at[pl.ds(sid * sub_rows + b * ZR, ZR)])
    pltpu.make_async_copy(src_hbm.at[wid], sidx, sem0).wait()
    pltpu.make_async_copy(dst_hbm.at[wid], didx, sem1).wait()
    plsc.subcore_barrier()

    pltpu.async_copy(g_hbm.at[sidx.at[0]], buf0, sem0)
    pltpu.async_copy(g_hbm.at[sidx.at[1]], buf1, sem1)

    def body(j, carry):
      i0 = 2 * j
      pltpu.make_async_copy(g_hbm.at[sidx.at[i0]], buf0, sem0).wait()
      pltpu.async_copy(buf0, acc_sh.at[didx.at[i0]], ssem0, add=True)
      pltpu.make_async_copy(g_hbm.at[sidx.at[i0 + 1]], buf1, sem1).wait()
      pltpu.async_copy(buf1, acc_sh.at[didx.at[i0 + 1]], ssem1, add=True)
      pltpu.make_async_copy(buf0, acc_sh.at[didx.at[i0]], ssem0).wait()
      pltpu.async_copy(g_hbm.at[sidx.at[i0 + 2]], buf0, sem0)
      pltpu.make_async_copy(buf1, acc_sh.at[didx.at[i0 + 1]], ssem1).wait()
      pltpu.async_copy(g_hbm.at[sidx.at[i0 + 3]], buf1, sem1)
      return carry

    lax.fori_loop(0, iters // 2 - 1, body, 0)
    i0 = iters - 2
    pltpu.make_async_copy(g_hbm.at[sidx.at[i0]], buf0, sem0).wait()
    pltpu.sync_copy(buf0, acc_sh.at[didx.at[i0]], add=True)
    pltpu.make_async_copy(g_hbm.at[sidx.at[i0 + 1]], buf1, sem1).wait()
    pltpu.sync_copy(buf1, acc_sh.at[didx.at[i0 + 1]], add=True)
    plsc.subcore_barrier()
    pltpu.sync_copy(
        acc_sh.at[pl.ds(sid * sub_rows, sub_rows)],
        out_hbm.at[cid].at[pl.ds(sid * sub_rows, sub_rows)],
    )

  return k(g, src3, dst3)


def _dinv(p0_ref, p1_ref):
  deg = p0_ref[...] + p1_ref[...] + 1.0
  return lax.rsqrt(deg)


def _tc_mm1(x, w1, p0, p1, blk):
  n, d_in = x.shape
  d_h = w1.shape[1]

  def body(x_ref, w_ref, p0_ref, p1_ref, o_ref):
    dinv = _dinv(p0_ref, p1_ref)
    h = jnp.dot(x_ref[...], w_ref[...], preferred_element_type=jnp.float32)
    o_ref[...] = h * dinv

  return pl.pallas_call(
      body,
      grid=(n // blk,),
      in_specs=[
          pl.BlockSpec((blk, d_in), lambda i: (i, 0)),
          pl.BlockSpec((d_in, d_h), lambda i: (0, 0)),
          pl.BlockSpec((blk, 1), lambda i: (i, 0)),
          pl.BlockSpec((blk, 1), lambda i: (i, 0)),
      ],
      out_specs=pl.BlockSpec((blk, d_h), lambda i: (i, 0)),
      out_shape=jax.ShapeDtypeStruct((n, d_h), jnp.float32),
  )(x, w1, p0, p1)


def _tc_mm2(a0, a1, g1, p0, p1, b1, w2, blk):
  n, d_h = g1.shape
  d_o = w2.shape[1]

  def body(a0_ref, a1_ref, g_ref, p0_ref, p1_ref, b_ref, w_ref, o_ref):
    dinv = _dinv(p0_ref, p1_ref)
    z = dinv * (a0_ref[...] + a1_ref[...] + g_ref[...]) + b_ref[...]
    h = jnp.maximum(z, 0.0)
    o_ref[...] = dinv * jnp.dot(
        h, w_ref[...], preferred_element_type=jnp.float32)

  return pl.pallas_call(
      body,
      grid=(n // blk,),
      in_specs=[
          pl.BlockSpec((blk, d_h), lambda i: (i, 0)),
          pl.BlockSpec((blk, d_h), lambda i: (i, 0)),
          pl.BlockSpec((blk, d_h), lambda i: (i, 0)),
          pl.BlockSpec((blk, 1), lambda i: (i, 0)),
          pl.BlockSpec((blk, 1), lambda i: (i, 0)),
          pl.BlockSpec((1, d_h), lambda i: (0, 0)),
          pl.BlockSpec((d_h, d_o), lambda i: (0, 0)),
      ],
      out_specs=pl.BlockSpec((blk, d_o), lambda i: (i, 0)),
      out_shape=jax.ShapeDtypeStruct((n, d_o), jnp.float32),
  )(a0, a1, g1, p0, p1, b1, w2)


def _tc_final(c0, c1, g2, p0, p1, b2, blk):
  n, d_o = g2.shape

  def body(c0_ref, c1_ref, g_ref, p0_ref, p1_ref, b_ref, z_ref, l_ref):
    dinv = _dinv(p0_ref, p1_ref)
    z = dinv * (c0_ref[...] + c1_ref[...] + g_ref[...]) + b_ref[...]
    z_ref[...] = z
    m = jnp.max(z, axis=1, keepdims=True)
    lse = jnp.log(jnp.sum(jnp.exp(z - m), axis=1, keepdims=True))
    l_ref[...] = z - m - lse

  return pl.pallas_call(
      body,
      grid=(n // blk,),
      in_specs=[
          pl.BlockSpec((blk, d_o), lambda i: (i, 0)),
          pl.BlockSpec((blk, d_o), lambda i: (i, 0)),
          pl.BlockSpec((blk, d_o), lambda i: (i, 0)),
          pl.BlockSpec((blk, 1), lambda i: (i, 0)),
          pl.BlockSpec((blk, 1), lambda i: (i, 0)),
          pl.BlockSpec((1, d_o), lambda i: (0, 0)),
      ],
      out_specs=[
          pl.BlockSpec((blk, d_o), lambda i: (i, 0)),
          pl.BlockSpec((blk, d_o), lambda i: (i, 0)),
      ],
      out_shape=[
          jax.ShapeDtypeStruct((n, d_o), jnp.float32),
          jax.ShapeDtypeStruct((n, d_o), jnp.float32),
      ],
  )(c0, c1, g2, p0, p1, b2)


@jax.jit
def kernel(x, edge_index, W1, b1, W2, b2):
  n, d_in = x.shape
  e = edge_index.shape[1]
  n_pad = ((n + NS * LANES - 1) // (NS * LANES)) * (NS * LANES)
  blk = 1024

  iters = (e + NW * C - 1) // (NW * C)
  if iters % 2:
    iters += 1
  e_pad = NW * C * iters
  pad_row = n + jnp.arange(e_pad - e, dtype=jnp.int32) % (n_pad - n)
  src3 = jnp.concatenate([edge_index[0], pad_row]).reshape(NW, iters, C)
  dst3 = jnp.concatenate([edge_index[1], pad_row]).reshape(NW, iters, C)
  xp = jnp.pad(x, ((0, n_pad - n), (0, 0)))

  degp = _sc_degree(dst3, n_pad, iters)
  p0 = degp[0][:, None]
  p1 = degp[1][:, None]

  g1 = _tc_mm1(xp, W1, p0, p1, blk)
  agg1 = _sc_aggregate(g1, src3, dst3, n_pad, iters, W1.shape[1])
  g2 = _tc_mm2(agg1[0], agg1[1], g1, p0, p1, b1[None, :], W2, blk)
  agg2 = _sc_aggregate(g2, src3, dst3, n_pad, iters, W2.shape[1])
  z2, lsm = _tc_final(agg2[0], agg2[1], g2, p0, p1, b2[None, :], blk)
  return (z2[:n], lsm[:n])


# speedup vs baseline: 1.1368x; 1.1368x over previous
import functools

import jax
import jax.numpy as jnp
from jax import lax
from jax.experimental import pallas as pl
from jax.experimental.pallas import tpu as pltpu
from jax.experimental.pallas import tpu_sc as plsc

NC = 2
NS = 16
NW = NC * NS
LANES = 16

C = 112
ZR = 64


def _sc_degree(dst3, n_pad, iters):
  sub_rows = n_pad // NS
  mesh = plsc.VectorSubcoreMesh(core_axis_name="c", subcore_axis_name="s")

  @functools.partial(
      pl.kernel,
      out_type=jax.ShapeDtypeStruct((NC, n_pad), jnp.float32),
      mesh=mesh,
      scratch_types=[
          pltpu.VMEM((iters, C), jnp.int32),
          pltpu.VMEM((C,), jnp.float32),
          pltpu.VMEM((sub_rows,), jnp.float32),
          pltpu.VMEM_SHARED((n_pad,), jnp.float32),
          pltpu.SemaphoreType.DMA,
      ],
  )
  def k(dst_hbm, out_hbm, idx_v, ones_v, zer_v, acc_sh, sem):
    cid = lax.axis_index("c")
    sid = lax.axis_index("s")
    wid = cid * NS + sid
    pltpu.async_copy(dst_hbm.at[wid], idx_v, sem)
    for i in range(C // LANES):
      ones_v[pl.ds(i * LANES, LANES)] = jnp.ones((LANES,), jnp.float32)
    for i in range(sub_rows // LANES):
      zer_v[pl.ds(i * LANES, LANES)] = jnp.zeros((LANES,), jnp.float32)
    pltpu.sync_copy(zer_v, acc_sh.at[pl.ds(sid * sub_rows, sub_rows)])
    pltpu.make_async_copy(dst_hbm.at[wid], idx_v, sem).wait()
    plsc.subcore_barrier()

    def body(i, carry):
      pltpu.sync_copy(ones_v, acc_sh.at[idx_v.at[i]], add=True)
      return carry

    lax.fori_loop(0, iters, body, 0)
    plsc.subcore_barrier()
    pltpu.sync_copy(
        acc_sh.at[pl.ds(sid * sub_rows, sub_rows)],
        out_hbm.at[cid, pl.ds(sid * sub_rows, sub_rows)],
    )

  return k(dst3)


def _sc_aggregate(g, src3, dst3, n_pad, iters, d):
  sub_rows = n_pad // NS
  mesh = plsc.VectorSubcoreMesh(core_axis_name="c", subcore_axis_name="s")

  @functools.partial(
      pl.kernel,
      out_type=jax.ShapeDtypeStruct((NC, n_pad, d), jnp.float32),
      mesh=mesh,
      scratch_types=[
          pltpu.VMEM((iters, C), jnp.int32),
          pltpu.VMEM((iters, C), jnp.int32),
          pltpu.VMEM((C, d), jnp.float32),
          pltpu.VMEM((C, d), jnp.float32),
          pltpu.VMEM_SHARED((n_pad, d), jnp.float32),
          pltpu.SemaphoreType.DMA,
          pltpu.SemaphoreType.DMA,
      ],
      compiler_params=pltpu.CompilerParams(use_tc_tiling_on_sc=False),
  )
  def k(g_hbm, src_hbm, dst_hbm, out_hbm, sidx, didx, buf0, buf1,
        acc_sh, sem0, sem1):
    cid = lax.axis_index("c")
    sid = lax.axis_index("s")
    wid = cid * NS + sid
    pltpu.async_copy(src_hbm.at[wid], sidx, sem0)
    pltpu.async_copy(dst_hbm.at[wid], didx, sem1)
    for r in range(ZR):
      for j in range(d // LANES):
        buf0[r, pl.ds(j * LANES, LANES)] = jnp.zeros((LANES,), jnp.float32)
    for b in range(sub_rows // ZR):
      pltpu.sync_copy(
          buf0.at[pl.ds(0, ZR)],
          acc_sh.at[pl.ds(sid * sub_rows + b * ZR, ZR)])
    pltpu.make_async_copy(src_hbm.at[wid], sidx, sem0).wait()
    pltpu.make_async_copy(dst_hbm.at[wid], didx, sem1).wait()
    plsc.subcore_barrier()

    pltpu.async_copy(g_hbm.at[sidx.at[0]], buf0, sem0)
    pltpu.async_copy(g_hbm.at[sidx.at[1]], buf1, sem1)

    def body(j, carry):
      i0 = 2 * j
      pltpu.make_async_copy(g_hbm.at[sidx.at[i0]], buf0, sem0).wait()
      pltpu.sync_copy(buf0, acc_sh.at[didx.at[i0]], add=True)
      pltpu.async_copy(g_hbm.at[sidx.at[i0 + 2]], buf0, sem0)
      pltpu.make_async_copy(g_hbm.at[sidx.at[i0 + 1]], buf1, sem1).wait()
      pltpu.sync_copy(buf1, acc_sh.at[didx.at[i0 + 1]], add=True)
      pltpu.async_copy(g_hbm.at[sidx.at[i0 + 3]], buf1, sem1)
      return carry

    lax.fori_loop(0, iters // 2 - 1, body, 0)
    i0 = iters - 2
    pltpu.make_async_copy(g_hbm.at[sidx.at[i0]], buf0, sem0).wait()
    pltpu.sync_copy(buf0, acc_sh.at[didx.at[i0]], add=True)
    pltpu.make_async_copy(g_hbm.at[sidx.at[i0 + 1]], buf1, sem1).wait()
    pltpu.sync_copy(buf1, acc_sh.at[didx.at[i0 + 1]], add=True)
    plsc.subcore_barrier()
    pltpu.sync_copy(
        acc_sh.at[pl.ds(sid * sub_rows, sub_rows)],
        out_hbm.at[cid].at[pl.ds(sid * sub_rows, sub_rows)],
    )

  return k(g, src3, dst3)


def _dinv(p0_ref, p1_ref):
  deg = p0_ref[...] + p1_ref[...] + 1.0
  return lax.rsqrt(deg)


def _tc_mm1(x, w1, p0, p1, blk):
  n, d_in = x.shape
  d_h = w1.shape[1]

  def body(x_ref, w_ref, p0_ref, p1_ref, o_ref):
    dinv = _dinv(p0_ref, p1_ref)
    h = jnp.dot(x_ref[...], w_ref[...], preferred_element_type=jnp.float32)
    o_ref[...] = h * dinv

  return pl.pallas_call(
      body,
      grid=(n // blk,),
      in_specs=[
          pl.BlockSpec((blk, d_in), lambda i: (i, 0)),
          pl.BlockSpec((d_in, d_h), lambda i: (0, 0)),
          pl.BlockSpec((blk, 1), lambda i: (i, 0)),
          pl.BlockSpec((blk, 1), lambda i: (i, 0)),
      ],
      out_specs=pl.BlockSpec((blk, d_h), lambda i: (i, 0)),
      out_shape=jax.ShapeDtypeStruct((n, d_h), jnp.float32),
  )(x, w1, p0, p1)


def _tc_mm2(a0, a1, g1, p0, p1, b1, w2, blk):
  n, d_h = g1.shape
  d_o = w2.shape[1]

  def body(a0_ref, a1_ref, g_ref, p0_ref, p1_ref, b_ref, w_ref, o_ref):
    dinv = _dinv(p0_ref, p1_ref)
    z = dinv * (a0_ref[...] + a1_ref[...] + g_ref[...]) + b_ref[...]
    h = jnp.maximum(z, 0.0)
    o_ref[...] = dinv * jnp.dot(
        h, w_ref[...], preferred_element_type=jnp.float32)

  return pl.pallas_call(
      body,
      grid=(n // blk,),
      in_specs=[
          pl.BlockSpec((blk, d_h), lambda i: (i, 0)),
          pl.BlockSpec((blk, d_h), lambda i: (i, 0)),
          pl.BlockSpec((blk, d_h), lambda i: (i, 0)),
          pl.BlockSpec((blk, 1), lambda i: (i, 0)),
          pl.BlockSpec((blk, 1), lambda i: (i, 0)),
          pl.BlockSpec((1, d_h), lambda i: (0, 0)),
          pl.BlockSpec((d_h, d_o), lambda i: (0, 0)),
      ],
      out_specs=pl.BlockSpec((blk, d_o), lambda i: (i, 0)),
      out_shape=jax.ShapeDtypeStruct((n, d_o), jnp.float32),
  )(a0, a1, g1, p0, p1, b1, w2)


def _tc_final(c0, c1, g2, p0, p1, b2, blk):
  n, d_o = g2.shape

  def body(c0_ref, c1_ref, g_ref, p0_ref, p1_ref, b_ref, z_ref, l_ref):
    dinv = _dinv(p0_ref, p1_ref)
    z = dinv * (c0_ref[...] + c1_ref[...] + g_ref[...]) + b_ref[...]
    z_ref[...] = z
    m = jnp.max(z, axis=1, keepdims=True)
    lse = jnp.log(jnp.sum(jnp.exp(z - m), axis=1, keepdims=True))
    l_ref[...] = z - m - lse

  return pl.pallas_call(
      body,
      grid=(n // blk,),
      in_specs=[
          pl.BlockSpec((blk, d_o), lambda i: (i, 0)),
          pl.BlockSpec((blk, d_o), lambda i: (i, 0)),
          pl.BlockSpec((blk, d_o), lambda i: (i, 0)),
          pl.BlockSpec((blk, 1), lambda i: (i, 0)),
          pl.BlockSpec((blk, 1), lambda i: (i, 0)),
          pl.BlockSpec((1, d_o), lambda i: (0, 0)),
      ],
      out_specs=[
          pl.BlockSpec((blk, d_o), lambda i: (i, 0)),
          pl.BlockSpec((blk, d_o), lambda i: (i, 0)),
      ],
      out_shape=[
          jax.ShapeDtypeStruct((n, d_o), jnp.float32),
          jax.ShapeDtypeStruct((n, d_o), jnp.float32),
      ],
  )(c0, c1, g2, p0, p1, b2)


@jax.jit
def kernel(x, edge_index, W1, b1, W2, b2):
  n, d_in = x.shape
  e = edge_index.shape[1]
  n_pad = ((n + NS * LANES - 1) // (NS * LANES)) * (NS * LANES)
  blk = 1024

  iters = (e + NW * C - 1) // (NW * C)
  if iters % 2:
    iters += 1
  e_pad = NW * C * iters
  pad_row = n + jnp.arange(e_pad - e, dtype=jnp.int32) % (n_pad - n)
  src3 = jnp.concatenate([edge_index[0], pad_row]).reshape(NW, iters, C)
  dst3 = jnp.concatenate([edge_index[1], pad_row]).reshape(NW, iters, C)
  xp = jnp.pad(x, ((0, n_pad - n), (0, 0)))

  degp = _sc_degree(dst3, n_pad, iters)
  p0 = degp[0][:, None]
  p1 = degp[1][:, None]

  g1 = _tc_mm1(xp, W1, p0, p1, blk)
  agg1 = _sc_aggregate(g1, src3, dst3, n_pad, iters, W1.shape[1])
  g2 = _tc_mm2(agg1[0], agg1[1], g1, p0, p1, b1[None, :], W2, blk)
  agg2 = _sc_aggregate(g2, src3, dst3, n_pad, iters, W2.shape[1])
  z2, lsm = _tc_final(agg2[0], agg2[1], g2, p0, p1, b2[None, :], blk)
  return (z2[:n], lsm[:n])


# speedup vs baseline: 1.1952x; 1.0514x over previous
import functools

import jax
import jax.numpy as jnp
from jax import lax
from jax.experimental import pallas as pl
from jax.experimental.pallas import tpu as pltpu
from jax.experimental.pallas import tpu_sc as plsc

NC = 2
NS = 16
NW = NC * NS
LANES = 16

C = 72
ZR = 64
RING = 12


def _sc_degree(dst3, n_pad, iters):
  sub_rows = n_pad // NS
  mesh = plsc.VectorSubcoreMesh(core_axis_name="c", subcore_axis_name="s")

  @functools.partial(
      pl.kernel,
      out_type=jax.ShapeDtypeStruct((NC, n_pad), jnp.float32),
      mesh=mesh,
      scratch_types=[
          pltpu.VMEM((iters, C), jnp.int32),
          pltpu.VMEM((((C + LANES - 1) // LANES) * LANES,), jnp.float32),
          pltpu.VMEM((sub_rows,), jnp.float32),
          pltpu.VMEM_SHARED((n_pad,), jnp.float32),
          pltpu.SemaphoreType.DMA,
      ],
  )
  def k(dst_hbm, out_hbm, idx_v, ones_v, zer_v, acc_sh, sem):
    cid = lax.axis_index("c")
    sid = lax.axis_index("s")
    wid = cid * NS + sid
    pltpu.async_copy(dst_hbm.at[wid], idx_v, sem)
    for i in range((C + LANES - 1) // LANES):
      ones_v[pl.ds(i * LANES, LANES)] = jnp.ones((LANES,), jnp.float32)
    for i in range(sub_rows // LANES):
      zer_v[pl.ds(i * LANES, LANES)] = jnp.zeros((LANES,), jnp.float32)
    pltpu.sync_copy(zer_v, acc_sh.at[pl.ds(sid * sub_rows, sub_rows)])
    pltpu.make_async_copy(dst_hbm.at[wid], idx_v, sem).wait()
    plsc.subcore_barrier()

    def body(i, carry):
      pltpu.sync_copy(
          ones_v.at[pl.ds(0, C)], acc_sh.at[idx_v.at[i]], add=True)
      return carry

    lax.fori_loop(0, iters, body, 0)
    plsc.subcore_barrier()
    pltpu.sync_copy(
        acc_sh.at[pl.ds(sid * sub_rows, sub_rows)],
        out_hbm.at[cid, pl.ds(sid * sub_rows, sub_rows)],
    )

  return k(dst3)


def _sc_aggregate(g, src3, dst3, n_pad, iters, d, nbuf):
  sub_rows = n_pad // NS
  mesh = plsc.VectorSubcoreMesh(core_axis_name="c", subcore_axis_name="s")

  @functools.partial(
      pl.kernel,
      out_type=jax.ShapeDtypeStruct((NC, n_pad, d), jnp.float32),
      mesh=mesh,
      scratch_types=(
          [pltpu.VMEM((iters, C), jnp.int32),
           pltpu.VMEM((iters, C), jnp.int32)]
          + [pltpu.VMEM((C, d), jnp.float32)] * nbuf
          + [pltpu.VMEM_SHARED((n_pad, d), jnp.float32)]
          + [pltpu.SemaphoreType.DMA] * nbuf
      ),
      compiler_params=pltpu.CompilerParams(use_tc_tiling_on_sc=False),
  )
  def k(g_hbm, src_hbm, dst_hbm, out_hbm, sidx, didx, *rest):
    bufs = rest[:nbuf]
    acc_sh = rest[nbuf]
    sems = rest[nbuf + 1:]
    cid = lax.axis_index("c")
    sid = lax.axis_index("s")
    wid = cid * NS + sid
    pltpu.async_copy(src_hbm.at[wid], sidx, sems[0])
    pltpu.async_copy(dst_hbm.at[wid], didx, sems[1 % nbuf])
    for r in range(ZR):
      for j in range(d // LANES):
        bufs[0][r, pl.ds(j * LANES, LANES)] = jnp.zeros((LANES,), jnp.float32)
    for b in range(sub_rows // ZR):
      pltpu.sync_copy(
          bufs[0].at[pl.ds(0, ZR)],
          acc_sh.at[pl.ds(sid * sub_rows + b * ZR, ZR)])
    pltpu.make_async_copy(src_hbm.at[wid], sidx, sems[0]).wait()
    pltpu.make_async_copy(dst_hbm.at[wid], didx, sems[1 % nbuf]).wait()
    plsc.subcore_barrier()

    for k_ in range(nbuf):
      pltpu.async_copy(g_hbm.at[sidx.at[k_]], bufs[k_], sems[k_])

    def body(j, carry):
      i0 = nbuf * j
      for k_ in range(nbuf):
        i = i0 + k_
        pltpu.make_async_copy(g_hbm.at[sidx.at[i]], bufs[k_], sems[k_]).wait()
        pltpu.sync_copy(bufs[k_], acc_sh.at[didx.at[i]], add=True)
        pltpu.async_copy(g_hbm.at[sidx.at[i + nbuf]], bufs[k_], sems[k_])
      return carry

    lax.fori_loop(0, iters // nbuf - 1, body, 0)
    i0 = iters - nbuf
    for k_ in range(nbuf):
      i = i0 + k_
      pltpu.make_async_copy(g_hbm.at[sidx.at[i]], bufs[k_], sems[k_]).wait()
      pltpu.sync_copy(bufs[k_], acc_sh.at[didx.at[i]], add=True)
    plsc.subcore_barrier()
    pltpu.sync_copy(
        acc_sh.at[pl.ds(sid * sub_rows, sub_rows)],
        out_hbm.at[cid].at[pl.ds(sid * sub_rows, sub_rows)],
    )

  return k(g, src3, dst3)


def _dinv(p0_ref, p1_ref):
  deg = p0_ref[...] + p1_ref[...] + 1.0
  return lax.rsqrt(deg)


def _tc_mm1(x, w1, p0, p1, blk):
  n, d_in = x.shape
  d_h = w1.shape[1]

  def body(x_ref, w_ref, p0_ref, p1_ref, o_ref):
    dinv = _dinv(p0_ref, p1_ref)
    h = jnp.dot(x_ref[...], w_ref[...], preferred_element_type=jnp.float32)
    o_ref[...] = h * dinv

  return pl.pallas_call(
      body,
      grid=(n // blk,),
      in_specs=[
          pl.BlockSpec((blk, d_in), lambda i: (i, 0)),
          pl.BlockSpec((d_in, d_h), lambda i: (0, 0)),
          pl.BlockSpec((blk, 1), lambda i: (i, 0)),
          pl.BlockSpec((blk, 1), lambda i: (i, 0)),
      ],
      out_specs=pl.BlockSpec((blk, d_h), lambda i: (i, 0)),
      out_shape=jax.ShapeDtypeStruct((n, d_h), jnp.float32),
  )(x, w1, p0, p1)


def _tc_mm2(a0, a1, g1, p0, p1, b1, w2, blk):
  n, d_h = g1.shape
  d_o = w2.shape[1]

  def body(a0_ref, a1_ref, g_ref, p0_ref, p1_ref, b_ref, w_ref, o_ref):
    dinv = _dinv(p0_ref, p1_ref)
    z = dinv * (a0_ref[...] + a1_ref[...] + g_ref[...]) + b_ref[...]
    h = jnp.maximum(z, 0.0)
    o_ref[...] = dinv * jnp.dot(
        h, w_ref[...], preferred_element_type=jnp.float32)

  return pl.pallas_call(
      body,
      grid=(n // blk,),
      in_specs=[
          pl.BlockSpec((blk, d_h), lambda i: (i, 0)),
          pl.BlockSpec((blk, d_h), lambda i: (i, 0)),
          pl.BlockSpec((blk, d_h), lambda i: (i, 0)),
          pl.BlockSpec((blk, 1), lambda i: (i, 0)),
          pl.BlockSpec((blk, 1), lambda i: (i, 0)),
          pl.BlockSpec((1, d_h), lambda i: (0, 0)),
          pl.BlockSpec((d_h, d_o), lambda i: (0, 0)),
      ],
      out_specs=pl.BlockSpec((blk, d_o), lambda i: (i, 0)),
      out_shape=jax.ShapeDtypeStruct((n, d_o), jnp.float32),
  )(a0, a1, g1, p0, p1, b1, w2)


def _tc_final(c0, c1, g2, p0, p1, b2, blk):
  n, d_o = g2.shape

  def body(c0_ref, c1_ref, g_ref, p0_ref, p1_ref, b_ref, z_ref, l_ref):
    dinv = _dinv(p0_ref, p1_ref)
    z = dinv * (c0_ref[...] + c1_ref[...] + g_ref[...]) + b_ref[...]
    z_ref[...] = z
    m = jnp.max(z, axis=1, keepdims=True)
    lse = jnp.log(jnp.sum(jnp.exp(z - m), axis=1, keepdims=True))
    l_ref[...] = z - m - lse

  return pl.pallas_call(
      body,
      grid=(n // blk,),
      in_specs=[
          pl.BlockSpec((blk, d_o), lambda i: (i, 0)),
          pl.BlockSpec((blk, d_o), lambda i: (i, 0)),
          pl.BlockSpec((blk, d_o), lambda i: (i, 0)),
          pl.BlockSpec((blk, 1), lambda i: (i, 0)),
          pl.BlockSpec((blk, 1), lambda i: (i, 0)),
          pl.BlockSpec((1, d_o), lambda i: (0, 0)),
      ],
      out_specs=[
          pl.BlockSpec((blk, d_o), lambda i: (i, 0)),
          pl.BlockSpec((blk, d_o), lambda i: (i, 0)),
      ],
      out_shape=[
          jax.ShapeDtypeStruct((n, d_o), jnp.float32),
          jax.ShapeDtypeStruct((n, d_o), jnp.float32),
      ],
  )(c0, c1, g2, p0, p1, b2)


@jax.jit
def kernel(x, edge_index, W1, b1, W2, b2):
  n, d_in = x.shape
  e = edge_index.shape[1]
  n_pad = ((n + NS * LANES - 1) // (NS * LANES)) * (NS * LANES)
  blk = 1024

  iters = (e + NW * C - 1) // (NW * C)
  iters = ((iters + RING - 1) // RING) * RING
  e_pad = NW * C * iters
  pad_row = n + jnp.arange(e_pad - e, dtype=jnp.int32) % (n_pad - n)
  src3 = jnp.concatenate([edge_index[0], pad_row]).reshape(NW, iters, C)
  dst3 = jnp.concatenate([edge_index[1], pad_row]).reshape(NW, iters, C)
  xp = jnp.pad(x, ((0, n_pad - n), (0, 0)))

  degp = _sc_degree(dst3, n_pad, iters)
  p0 = degp[0][:, None]
  p1 = degp[1][:, None]

  g1 = _tc_mm1(xp, W1, p0, p1, blk)
  agg1 = _sc_aggregate(g1, src3, dst3, n_pad, iters, W1.shape[1], 3)
  g2 = _tc_mm2(agg1[0], agg1[1], g1, p0, p1, b1[None, :], W2, blk)
  agg2 = _sc_aggregate(g2, src3, dst3, n_pad, iters, W2.shape[1], 4)
  z2, lsm = _tc_final(agg2[0], agg2[1], g2, p0, p1, b2[None, :], blk)
  return (z2[:n], lsm[:n])
